# baseline (speedup 1.0000x reference)
import functools

import jax
import jax.numpy as jnp
from jax import lax
from jax.experimental import pallas as pl
from jax.experimental.pallas import tpu as pltpu
from jax.experimental.pallas import tpu_sc as plsc

NC = 2
NS = 16
NW = NC * NS
LANES = 16

SEQ = 200
EMB = 64
BCOL = 128


@jax.jit
def _fused_embed(xT, table, pe200):
    mesh = plsc.VectorSubcoreMesh(core_axis_name="c", subcore_axis_name="s")

    @functools.partial(
        pl.kernel,
        out_type=jax.ShapeDtypeStruct((SEQ, 8, NW, 8, BCOL), jnp.float32),
        mesh=mesh,
        scratch_types=[
            pltpu.VMEM((SEQ, BCOL), jnp.int32),
            pltpu.VMEM((SEQ, EMB), jnp.float32),
            pltpu.VMEM((3, BCOL, EMB), jnp.float32),
            pltpu.VMEM((2, 8, 8, BCOL), jnp.float32),
            pltpu.SemaphoreType.DMA,
            pltpu.SemaphoreType.DMA,
            pltpu.SemaphoreType.DMA,
            pltpu.SemaphoreType.DMA,
            pltpu.SemaphoreType.DMA,
        ],
        compiler_params=pltpu.CompilerParams(
            needs_layout_passes=False, use_tc_tiling_on_sc=False),
    )
    def body(xT_hbm, tab_hbm, pe_hbm, out_hbm,
             idx_v, pe_v, rows_v, tout_v,
             gsem0, gsem1, gsem2, wsem0, wsem1):
        wid = lax.axis_index("s") * NC + lax.axis_index("c")
        bcol = wid * BCOL
        iota = lax.iota(jnp.int32, LANES)

        pltpu.sync_copy(xT_hbm.at[slice(None), pl.ds(bcol, BCOL)], idx_v)
        pltpu.sync_copy(pe_hbm, pe_v)

        gsems = (gsem0, gsem1, gsem2)
        wsems = (wsem0, wsem1)
        for p in range(3):
            pltpu.async_copy(tab_hbm.at[idx_v.at[p]], rows_v.at[p], gsems[p])

        def out_slice(s):
            return out_hbm.at[s, slice(None), wid, slice(None), slice(None)]

        def do_step(s, q):
            static = isinstance(s, int)
            rv, tv = rows_v.at[q % 3], tout_v.at[q % 2]
            gs, ws = gsems[q % 3], wsems[q % 2]
            pltpu.make_async_copy(tab_hbm.at[idx_v.at[s]], rv, gs).wait()

            def wait_w():
                pltpu.make_async_copy(tv, out_slice(s - 2), ws).wait()

            if static:
                wait_w()
            else:
                pl.when(s >= 2)(wait_w)

            spl = jnp.full((LANES,), s, jnp.int32)

            @plsc.parallel_loop(0, EMB, unroll=8)
            def _(d):
                cidx = jnp.full((LANES,), d, jnp.int32)
                pev = plsc.load_gather(pe_v, [spl, cidx])
                for j in range(BCOL // LANES):
                    vals = plsc.load_gather(rv, [iota + (j * LANES), cidx])
                    tv[d // 8, d % 8, pl.ds(j * LANES, LANES)] = vals + pev

            def prefetch():
                pltpu.async_copy(tab_hbm.at[idx_v.at[s + 3]], rv, gs)

            if static:
                if s + 3 < SEQ:
                    prefetch()
            else:
                pl.when(s + 3 < SEQ)(prefetch)

            pltpu.async_copy(tv, out_slice(s), ws)

        def step(i, carry):
            for q in range(6):
                do_step(i * 6 + q, q)
            return carry

        lax.fori_loop(0, (SEQ - 2) // 6, step, 0)
        do_step(SEQ - 2, 0)
        do_step(SEQ - 1, 1)
        for p in range(2):
            pltpu.make_async_copy(
                tout_v.at[p], out_slice(SEQ - 2 + p), wsems[p]).wait()

    return body(xT, table, pe200)


def kernel(x, table, pe):
    batch, seq = x.shape
    emb_dim = table.shape[1]
    xT = x.T
    out5 = _fused_embed(xT, table, pe[:seq])
    r = out5.transpose(0, 1, 3, 2, 4).reshape(seq, emb_dim, batch)
    return r.transpose(2, 0, 1)


# speedup vs baseline: 1.0001x; 1.0001x over previous
import functools

import jax
import jax.numpy as jnp
from jax import lax
from jax.experimental import pallas as pl
from jax.experimental.pallas import tpu as pltpu
from jax.experimental.pallas import tpu_sc as plsc

NC = 2
NS = 16
NW = NC * NS
LANES = 16

SEQ = 200
EMB = 64
BCOL = 128
SB = 2
NSS = SEQ // SB


@jax.jit
def _fused_embed(xT, table, pe200):
    mesh = plsc.VectorSubcoreMesh(core_axis_name="c", subcore_axis_name="s")

    @functools.partial(
        pl.kernel,
        out_type=jax.ShapeDtypeStruct((SEQ, 8, NW, 8, BCOL), jnp.float32),
        mesh=mesh,
        scratch_types=[
            pltpu.VMEM((SEQ * BCOL,), jnp.int32),
            pltpu.VMEM((SEQ, EMB), jnp.float32),
            pltpu.VMEM((2, SB * BCOL, EMB), jnp.float32),
            pltpu.VMEM((2, SB, 8, 8, BCOL), jnp.float32),
            pltpu.SemaphoreType.DMA,
            pltpu.SemaphoreType.DMA,
            pltpu.SemaphoreType.DMA,
            pltpu.SemaphoreType.DMA,
        ],
        compiler_params=pltpu.CompilerParams(
            needs_layout_passes=False, use_tc_tiling_on_sc=False),
    )
    def body(xr_hbm, tab_hbm, pe_hbm, out_hbm,
             idx_v, pe_v, rows_v, tout_v,
             gsem0, gsem1, wsem0, wsem1):
        wid = lax.axis_index("s") * NC + lax.axis_index("c")
        iota = lax.iota(jnp.int32, LANES)

        pltpu.sync_copy(xr_hbm.at[wid], idx_v)
        pltpu.sync_copy(pe_hbm, pe_v)

        gsems = (gsem0, gsem1)
        wsems = (wsem0, wsem1)

        def idx_slice(k):
            return idx_v.at[pl.ds(k * (SB * BCOL), SB * BCOL)]

        def out_slice(k):
            return out_hbm.at[
                pl.ds(k * SB, SB), slice(None), wid, slice(None), slice(None)]

        for p in range(2):
            pltpu.async_copy(tab_hbm.at[idx_slice(p)], rows_v.at[p], gsems[p])

        def do_step(k, q):
            rv, tv = rows_v.at[q], tout_v.at[q]
            gs, ws = gsems[q], wsems[q]
            pltpu.make_async_copy(tab_hbm.at[idx_slice(k)], rv, gs).wait()

            @pl.when(k >= 2)
            def _():
                pltpu.make_async_copy(tv, out_slice(k - 2), ws).wait()

            for sl in range(SB):
                s = k * SB + sl
                spl = jnp.full((LANES,), s, jnp.int32)
                rvs, tvs = rv.at[pl.ds(sl * BCOL, BCOL)], tv.at[sl]

                @plsc.parallel_loop(0, EMB, unroll=8)
                def _(d):
                    cidx = jnp.full((LANES,), d, jnp.int32)
                    pev = plsc.load_gather(pe_v, [spl, cidx])
                    for j in range(BCOL // LANES):
                        vals = plsc.load_gather(rvs, [iota + (j * LANES), cidx])
                        tvs[d // 8, d % 8, pl.ds(j * LANES, LANES)] = vals + pev

            @pl.when(k + 2 < NSS)
            def _():
                pltpu.async_copy(tab_hbm.at[idx_slice(k + 2)], rv, gs)

            pltpu.async_copy(tv, out_slice(k), ws)

        def step(i, carry):
            for q in range(2):
                do_step(i * 2 + q, q)
            return carry

        lax.fori_loop(0, NSS // 2, step, 0)
        for p in range(2):
            pltpu.make_async_copy(
                tout_v.at[p], out_slice(NSS - 2 + p), wsems[p]).wait()

    return body(xT, table, pe200)


def kernel(x, table, pe):
    batch, seq = x.shape
    emb_dim = table.shape[1]
    xr = x.reshape(NW, BCOL, NSS, SB).transpose(0, 2, 3, 1).reshape(NW, seq * BCOL)
    out5 = _fused_embed(xr, table, pe[:seq])
    r = out5.transpose(0, 1, 3, 2, 4).reshape(seq, emb_dim, batch)
    return r.transpose(2, 0, 1)


# speedup vs baseline: 1.0187x; 1.0186x over previous
import functools

import jax
import jax.numpy as jnp
from jax import lax
from jax.experimental import pallas as pl
from jax.experimental.pallas import tpu as pltpu
from jax.experimental.pallas import tpu_sc as plsc

NC = 2
NS = 16
NW = NC * NS
LANES = 16

SEQ = 200
EMB = 64
BCOL = 128
SB = 2
NSS = SEQ // SB


@jax.jit
def _fused_embed(xT, table, pe200):
    mesh = plsc.VectorSubcoreMesh(core_axis_name="c", subcore_axis_name="s")

    @functools.partial(
        pl.kernel,
        out_type=jax.ShapeDtypeStruct((SEQ, 8, NW, 8, BCOL), jnp.float32),
        mesh=mesh,
        scratch_types=[
            pltpu.VMEM((SEQ * BCOL,), jnp.int32),
            pltpu.VMEM((SEQ, EMB), jnp.float32),
            pltpu.VMEM((2, SB * BCOL, EMB), jnp.float32),
            pltpu.VMEM((2, SB, 8, 8, BCOL), jnp.float32),
            pltpu.SemaphoreType.DMA,
            pltpu.SemaphoreType.DMA,
            pltpu.SemaphoreType.DMA,
            pltpu.SemaphoreType.DMA,
        ],
        compiler_params=pltpu.CompilerParams(
            needs_layout_passes=False, use_tc_tiling_on_sc=False),
    )
    def body(xr_hbm, tab_hbm, pe_hbm, out_hbm,
             idx_v, pe_v, rows_v, tout_v,
             gsem0, gsem1, wsem0, wsem1):
        wid = lax.axis_index("s") * NC + lax.axis_index("c")
        iota = lax.iota(jnp.int32, LANES)

        pltpu.sync_copy(xr_hbm.at[wid], idx_v)
        pltpu.sync_copy(pe_hbm, pe_v)

        gsems = (gsem0, gsem1)
        wsems = (wsem0, wsem1)

        def idx_slice(k):
            return idx_v.at[pl.ds(k * (SB * BCOL), SB * BCOL)]

        def out_slice(k):
            return out_hbm.at[
                pl.ds(k * SB, SB), slice(None), wid, slice(None), slice(None)]

        for p in range(2):
            pltpu.async_copy(tab_hbm.at[idx_slice(p)], rows_v.at[p], gsems[p])

        def do_step(k, q):
            rv, tv = rows_v.at[q], tout_v.at[q]
            gs, ws = gsems[q], wsems[q]
            pltpu.make_async_copy(tab_hbm.at[idx_slice(k)], rv, gs).wait()

            @pl.when(k >= 2)
            def _():
                pltpu.make_async_copy(tv, out_slice(k - 2), ws).wait()

            for sl in range(SB):
                s = k * SB + sl
                spl = jnp.full((LANES,), s, jnp.int32)
                rvs, tvs = rv.at[pl.ds(sl * BCOL, BCOL)], tv.at[sl]

                @plsc.parallel_loop(0, EMB, unroll=16)
                def _(d):
                    cidx = jnp.full((LANES,), d, jnp.int32)
                    pev = plsc.load_gather(pe_v, [spl, cidx])
                    for j in range(BCOL // LANES):
                        vals = plsc.load_gather(rvs, [iota + (j * LANES), cidx])
                        tvs[d // 8, d % 8, pl.ds(j * LANES, LANES)] = vals + pev

            @pl.when(k + 2 < NSS)
            def _():
                pltpu.async_copy(tab_hbm.at[idx_slice(k + 2)], rv, gs)

            pltpu.async_copy(tv, out_slice(k), ws)

        def step(i, carry):
            for q in range(2):
                do_step(i * 2 + q, q)
            return carry

        lax.fori_loop(0, NSS // 2, step, 0)
        for p in range(2):
            pltpu.make_async_copy(
                tout_v.at[p], out_slice(NSS - 2 + p), wsems[p]).wait()

    return body(xT, table, pe200)


def kernel(x, table, pe):
    batch, seq = x.shape
    emb_dim = table.shape[1]
    xr = x.reshape(NW, BCOL, NSS, SB).transpose(0, 2, 3, 1).reshape(NW, seq * BCOL)
    out5 = _fused_embed(xr, table, pe[:seq])
    r = out5.transpose(0, 1, 3, 2, 4).reshape(seq, emb_dim, batch)
    return r.transpose(2, 0, 1)


# speedup vs baseline: 1.7927x; 1.7598x over previous
import functools

import jax
import jax.numpy as jnp
from jax import lax
from jax.experimental import pallas as pl
from jax.experimental.pallas import tpu as pltpu
from jax.experimental.pallas import tpu_sc as plsc

NC = 2
NS = 16
NW = NC * NS
LANES = 16

SEQ = 200
EMB = 64
BCOL = 128
SB = 2
NSS = SEQ // SB


@jax.jit
def _fused_embed(xT, table, pe200):
    mesh = plsc.VectorSubcoreMesh(core_axis_name="c", subcore_axis_name="s")

    @functools.partial(
        pl.kernel,
        out_type=jax.ShapeDtypeStruct((SEQ, 8, NW, 8, BCOL), jnp.float32),
        mesh=mesh,
        scratch_types=[
            pltpu.VMEM((SEQ * BCOL,), jnp.int32),
            pltpu.VMEM((SEQ, EMB), jnp.float32),
            pltpu.VMEM((2, SB * BCOL, EMB), jnp.float32),
            pltpu.VMEM((2, SB, 8, 8, BCOL + 1), jnp.float32),
            pltpu.SemaphoreType.DMA,
            pltpu.SemaphoreType.DMA,
            pltpu.SemaphoreType.DMA,
            pltpu.SemaphoreType.DMA,
        ],
        compiler_params=pltpu.CompilerParams(
            needs_layout_passes=False, use_tc_tiling_on_sc=False),
    )
    def body(xr_hbm, tab_hbm, pe_hbm, out_hbm,
             idx_v, pe_v, rows_v, tout_v,
             gsem0, gsem1, wsem0, wsem1):
        wid = lax.axis_index("s") * NC + lax.axis_index("c")
        iota = lax.iota(jnp.int32, LANES)

        pltpu.sync_copy(xr_hbm.at[wid], idx_v)
        pltpu.sync_copy(pe_hbm, pe_v)

        gsems = (gsem0, gsem1)
        wsems = (wsem0, wsem1)

        def idx_slice(k):
            return idx_v.at[pl.ds(k * (SB * BCOL), SB * BCOL)]

        def out_slice(k):
            return out_hbm.at[
                pl.ds(k * SB, SB), slice(None), wid, slice(None), slice(None)]

        for p in range(2):
            pltpu.async_copy(tab_hbm.at[idx_slice(p)], rows_v.at[p], gsems[p])

        def do_step(k, q):
            rv, tv = rows_v.at[q], tout_v.at[q]
            gs, ws = gsems[q], wsems[q]
            pltpu.make_async_copy(tab_hbm.at[idx_slice(k)], rv, gs).wait()

            @pl.when(k >= 2)
            def _():
                pltpu.make_async_copy(
                    tv.at[slice(None), slice(None), slice(None), pl.ds(0, BCOL)],
                    out_slice(k - 2), ws).wait()

            for sl in range(SB):
                s = k * SB + sl
                rvs, tvs = rv.at[pl.ds(sl * BCOL, BCOL)], tv.at[sl]
                pec = [pe_v[s, pl.ds(c * LANES, LANES)] for c in range(4)]

                @plsc.parallel_loop(0, BCOL, unroll=16)
                def _(b):
                    bspl = jnp.full((LANES,), b, jnp.int32)
                    for c in range(4):
                        d16 = (c * LANES) + iota
                        vals = rvs[b, pl.ds(c * LANES, LANES)] + pec[c]
                        plsc.store_scatter(tvs, [d16 // 8, d16 % 8, bspl], vals)

            @pl.when(k + 2 < NSS)
            def _():
                pltpu.async_copy(tab_hbm.at[idx_slice(k + 2)], rv, gs)

            pltpu.async_copy(
                tv.at[slice(None), slice(None), slice(None), pl.ds(0, BCOL)],
                out_slice(k), ws)

        def step(i, carry):
            for q in range(2):
                do_step(i * 2 + q, q)
            return carry

        lax.fori_loop(0, NSS // 2, step, 0)
        for p in range(2):
            pltpu.make_async_copy(
                tout_v.at[p, slice(None), slice(None), slice(None), pl.ds(0, BCOL)],
                out_slice(NSS - 2 + p), wsems[p]).wait()

    return body(xT, table, pe200)


def kernel(x, table, pe):
    batch, seq = x.shape
    emb_dim = table.shape[1]
    xr = x.reshape(NW, BCOL, NSS, SB).transpose(0, 2, 3, 1).reshape(NW, seq * BCOL)
    out5 = _fused_embed(xr, table, pe[:seq])
    r = out5.transpose(0, 1, 3, 2, 4).reshape(seq, emb_dim, batch)
    return r.transpose(2, 0, 1)
